# baseline (speedup 1.0000x reference)
import functools

import jax
import jax.numpy as jnp
from jax import lax
from jax.experimental import pallas as pl
from jax.experimental.pallas import tpu as pltpu
from jax.experimental.pallas import tpu_sc as plsc

VOCAB = 1000000
EMBED = 32
NUM_CLASSES = 2
BATCH = 16384

NC = 2
NS = 16
NW = NC * NS
B_PER_W = BATCH // NW
GCH = 128
L = 16

BN = 98304


def _mm_body(x_ref, w_ref, b_ref, o0_ref, o1_ref):
    x = x_ref[...]
    w = w_ref[...]
    b = b_ref[...]
    y = lax.dot_general(w, x, (((1,), (0,)), ((), ())),
                        preferred_element_type=jnp.float32,
                        precision=lax.Precision.DEFAULT)
    o0_ref[...] = y[0] + b[0, 0]
    o1_ref[...] = y[1] + b[0, 1]


def _tc_vocab_linear(tabT, W, b2):
    grid = (VOCAB + BN - 1) // BN
    return pl.pallas_call(
        _mm_body,
        grid=(grid,),
        in_specs=[
            pl.BlockSpec((EMBED, BN), lambda i: (0, i)),
            pl.BlockSpec((NUM_CLASSES, EMBED), lambda i: (0, 0)),
            pl.BlockSpec((1, NUM_CLASSES), lambda i: (0, 0)),
        ],
        out_specs=[
            pl.BlockSpec((BN,), lambda i: (i,)),
            pl.BlockSpec((BN,), lambda i: (i,)),
        ],
        out_shape=[
            jax.ShapeDtypeStruct((VOCAB,), jnp.float32),
            jax.ShapeDtypeStruct((VOCAB,), jnp.float32),
        ],
    )(tabT, W, b2)


def _sc_gather_planes(P0, P1, word_ids):
    mesh = plsc.VectorSubcoreMesh(core_axis_name="c", subcore_axis_name="s")

    @functools.partial(
        pl.kernel,
        mesh=mesh,
        out_type=[
            jax.ShapeDtypeStruct((BATCH,), jnp.float32),
            jax.ShapeDtypeStruct((BATCH,), jnp.float32),
        ],
        compiler_params=pltpu.CompilerParams(needs_layout_passes=False),
        scratch_types=[
            pltpu.VMEM((B_PER_W,), jnp.int32),
            pltpu.VMEM((B_PER_W,), jnp.float32),
            pltpu.VMEM((B_PER_W,), jnp.float32),
            pltpu.SemaphoreType.DMA,
        ],
    )
    def k(p0_hbm, p1_hbm, idx_hbm, out0_hbm, out1_hbm,
          idx_v, vals0_v, vals1_v, sem):
        wid = lax.axis_index("s") * NC + lax.axis_index("c")
        base = wid * B_PER_W
        pltpu.sync_copy(idx_hbm.at[pl.ds(base, B_PER_W)], idx_v)

        cps = []
        for q in range(B_PER_W // GCH):
            sl = pl.ds(q * GCH, GCH)
            cps.append(pltpu.async_copy(
                p0_hbm.at[idx_v.at[sl]], vals0_v.at[sl], sem))
            cps.append(pltpu.async_copy(
                p1_hbm.at[idx_v.at[sl]], vals1_v.at[sl], sem))
        for cp in cps:
            cp.wait()

        pltpu.sync_copy(vals0_v, out0_hbm.at[pl.ds(base, B_PER_W)])
        pltpu.sync_copy(vals1_v, out1_hbm.at[pl.ds(base, B_PER_W)])

    return k(P0, P1, word_ids)


def kernel(word_ids, embedding, W, b):
    tabT = embedding.T
    P0, P1 = _tc_vocab_linear(tabT, W, b.reshape(1, NUM_CLASSES))
    o0, o1 = _sc_gather_planes(P0, P1, word_ids)
    return jnp.stack([o0, o1], axis=1)


# speedup vs baseline: 1.0173x; 1.0173x over previous
import functools

import jax
import jax.numpy as jnp
from jax import lax
from jax.experimental import pallas as pl
from jax.experimental.pallas import tpu as pltpu
from jax.experimental.pallas import tpu_sc as plsc

VOCAB = 1000000
EMBED = 32
NUM_CLASSES = 2
BATCH = 16384

NC = 2
NS = 16
NW = NC * NS
B_PER_W = BATCH // NW
GCH = 128
L = 16

BN = 65536


def _mm_body(x_ref, w_ref, b_ref, o0_ref, o1_ref):
    x = x_ref[...]
    w = w_ref[...]
    b = b_ref[...]
    y = lax.dot_general(w, x, (((1,), (0,)), ((), ())),
                        preferred_element_type=jnp.float32,
                        precision=lax.Precision.DEFAULT)
    o0_ref[...] = y[0] + b[0, 0]
    o1_ref[...] = y[1] + b[0, 1]


def _tc_vocab_linear(tabT, W, b2):
    grid = (VOCAB + BN - 1) // BN
    return pl.pallas_call(
        _mm_body,
        grid=(grid,),
        in_specs=[
            pl.BlockSpec((EMBED, BN), lambda i: (0, i)),
            pl.BlockSpec((NUM_CLASSES, EMBED), lambda i: (0, 0)),
            pl.BlockSpec((1, NUM_CLASSES), lambda i: (0, 0)),
        ],
        out_specs=[
            pl.BlockSpec((BN,), lambda i: (i,)),
            pl.BlockSpec((BN,), lambda i: (i,)),
        ],
        out_shape=[
            jax.ShapeDtypeStruct((VOCAB,), jnp.float32),
            jax.ShapeDtypeStruct((VOCAB,), jnp.float32),
        ],
    )(tabT, W, b2)


def _sc_gather_planes(P0, P1, word_ids):
    mesh = plsc.VectorSubcoreMesh(core_axis_name="c", subcore_axis_name="s")

    @functools.partial(
        pl.kernel,
        mesh=mesh,
        out_type=jax.ShapeDtypeStruct((BATCH * NUM_CLASSES,), jnp.float32),
        compiler_params=pltpu.CompilerParams(needs_layout_passes=False),
        scratch_types=[
            pltpu.VMEM((B_PER_W,), jnp.int32),
            pltpu.VMEM((B_PER_W,), jnp.float32),
            pltpu.VMEM((B_PER_W,), jnp.float32),
            pltpu.SemaphoreType.DMA,
        ],
    )
    def k(p0_hbm, p1_hbm, idx_hbm, out_hbm,
          idx_v, vals0_v, vals1_v, sem):
        wid = lax.axis_index("s") * NC + lax.axis_index("c")
        base = wid * B_PER_W
        pltpu.sync_copy(idx_hbm.at[pl.ds(base, B_PER_W)], idx_v)

        cps = []
        for q in range(B_PER_W // GCH):
            sl = pl.ds(q * GCH, GCH)
            cps.append(pltpu.async_copy(
                p0_hbm.at[idx_v.at[sl]], vals0_v.at[sl], sem))
            cps.append(pltpu.async_copy(
                p1_hbm.at[idx_v.at[sl]], vals1_v.at[sl], sem))
        for cp in cps:
            cp.wait()

        pltpu.sync_copy(vals0_v, out_hbm.at[pl.ds(base, B_PER_W)])
        pltpu.sync_copy(vals1_v, out_hbm.at[pl.ds(BATCH + base, B_PER_W)])

    return k(P0, P1, word_ids)


def kernel(word_ids, embedding, W, b):
    tabT = embedding.T
    P0, P1 = _tc_vocab_linear(tabT, W, b.reshape(1, NUM_CLASSES))
    flat = _sc_gather_planes(P0, P1, word_ids)
    return flat.reshape(NUM_CLASSES, BATCH).T


# speedup vs baseline: 1.0221x; 1.0047x over previous
import functools

import jax
import jax.numpy as jnp
from jax import lax
from jax.experimental import pallas as pl
from jax.experimental.pallas import tpu as pltpu
from jax.experimental.pallas import tpu_sc as plsc

VOCAB = 1000000
EMBED = 32
NUM_CLASSES = 2
BATCH = 16384

NC = 2
NS = 16
NW = NC * NS
B_PER_W = BATCH // NW
GCH = 128
L = 16

BN = 65536


def _mm_body(x_ref, w_ref, b_ref, o0_ref, o1_ref):
    x = x_ref[...]
    w = w_ref[...]
    b = b_ref[...]
    y = lax.dot_general(w, x, (((1,), (0,)), ((), ())),
                        preferred_element_type=jnp.float32,
                        precision=lax.Precision.DEFAULT)
    o0_ref[...] = y[0] + b[0, 0]
    o1_ref[...] = y[1] + b[0, 1]


def _tc_vocab_linear(tabT, W, b2):
    grid = (VOCAB + BN - 1) // BN
    return pl.pallas_call(
        _mm_body,
        grid=(grid,),
        in_specs=[
            pl.BlockSpec((EMBED, BN), lambda i: (0, i),
                         pipeline_mode=pl.Buffered(buffer_count=2)),
            pl.BlockSpec((NUM_CLASSES, EMBED), lambda i: (0, 0)),
            pl.BlockSpec((1, NUM_CLASSES), lambda i: (0, 0)),
        ],
        out_specs=[
            pl.BlockSpec((BN,), lambda i: (i,)),
            pl.BlockSpec((BN,), lambda i: (i,)),
        ],
        out_shape=[
            jax.ShapeDtypeStruct((VOCAB,), jnp.float32),
            jax.ShapeDtypeStruct((VOCAB,), jnp.float32),
        ],
    )(tabT, W, b2)


def _sc_gather_planes(P0, P1, word_ids):
    mesh = plsc.VectorSubcoreMesh(core_axis_name="c", subcore_axis_name="s")

    @functools.partial(
        pl.kernel,
        mesh=mesh,
        out_type=jax.ShapeDtypeStruct((BATCH * NUM_CLASSES,), jnp.float32),
        compiler_params=pltpu.CompilerParams(needs_layout_passes=False),
        scratch_types=[
            pltpu.VMEM((B_PER_W,), jnp.int32),
            pltpu.VMEM((B_PER_W,), jnp.float32),
            pltpu.VMEM((B_PER_W,), jnp.float32),
            pltpu.SemaphoreType.DMA,
        ],
    )
    def k(p0_hbm, p1_hbm, idx_hbm, out_hbm,
          idx_v, vals0_v, vals1_v, sem):
        wid = lax.axis_index("s") * NC + lax.axis_index("c")
        base = wid * B_PER_W
        pltpu.sync_copy(idx_hbm.at[pl.ds(base, B_PER_W)], idx_v)

        cps = []
        for q in range(B_PER_W // GCH):
            sl = pl.ds(q * GCH, GCH)
            cps.append(pltpu.async_copy(
                p0_hbm.at[idx_v.at[sl]], vals0_v.at[sl], sem))
            cps.append(pltpu.async_copy(
                p1_hbm.at[idx_v.at[sl]], vals1_v.at[sl], sem))
        for cp in cps:
            cp.wait()

        pltpu.sync_copy(vals0_v, out_hbm.at[pl.ds(base, B_PER_W)])
        pltpu.sync_copy(vals1_v, out_hbm.at[pl.ds(BATCH + base, B_PER_W)])

    return k(P0, P1, word_ids)


def kernel(word_ids, embedding, W, b):
    tabT = embedding.T
    P0, P1 = _tc_vocab_linear(tabT, W, b.reshape(1, NUM_CLASSES))
    flat = _sc_gather_planes(P0, P1, word_ids)
    return flat.reshape(NUM_CLASSES, BATCH).T
